# baseline (speedup 1.0000x reference)
import dataclasses
import functools

import jax
import jax.numpy as jnp
import numpy as np
from jax import lax
from jax.experimental import pallas as pl
from jax.experimental.pallas import tpu as pltpu
from jax.experimental.pallas import tpu_sc as plsc

NC, NS, L = 2, 16, 16
NW = NC * NS


def _fm_sc(idx_flat, w1d, vp, b16, *, B, F, N, K):
    EPW = B // NW
    IPW = EPW * F
    E_CH = 64
    ROWS = E_CH * F
    NCHUNK = EPW // E_CH
    NGRP = E_CH // L

    mesh = plsc.VectorSubcoreMesh(
        core_axis_name="c", subcore_axis_name="s", num_cores=NC, num_subcores=NS
    )

    cp = pltpu.CompilerParams(use_tc_tiling_on_sc=False)
    if "needs_layout_passes" in pltpu.CompilerParams.__dataclass_fields__:
        cp = dataclasses.replace(cp, needs_layout_passes=False)

    @functools.partial(
        pl.kernel,
        out_type=jax.ShapeDtypeStruct((B,), jnp.float32),
        mesh=mesh,
        compiler_params=cp,
        scratch_types=[
            pltpu.VMEM((IPW,), jnp.int32),
            pltpu.VMEM((ROWS, K), jnp.float32),
            pltpu.VMEM((ROWS, K), jnp.float32),
            pltpu.VMEM((ROWS + L,), jnp.float32),
            pltpu.VMEM((ROWS + L,), jnp.float32),
            pltpu.VMEM((EPW,), jnp.float32),
            pltpu.VMEM((EPW,), jnp.float32),
            pltpu.VMEM((L,), jnp.float32),
            pltpu.SemaphoreType.DMA,
            pltpu.SemaphoreType.DMA,
            pltpu.SemaphoreType.DMA,
            pltpu.SemaphoreType.DMA,
        ],
    )
    def fm_kernel(idx_hbm, w_hbm, v_hbm, b_hbm, out_hbm,
                  idxv, g0, g1, w0, w1, tbuf, obuf, bbuf,
                  sv0, sv1, sw0, sw1):
        wid = lax.axis_index("s") * NC + lax.axis_index("c")

        pltpu.sync_copy(idx_hbm.at[pl.ds(wid * IPW, IPW)], idxv)
        pltpu.sync_copy(b_hbm, bbuf)

        gbufs, wbufs, svs, sws = (g0, g1), (w0, w1), (sv0, sv1), (sw0, sw1)

        def start_gather(g):
            sel = g % 2
            idx_slice = idxv.at[pl.ds(g * ROWS, ROWS)]
            cv = pltpu.async_copy(v_hbm.at[idx_slice], gbufs[sel], svs[sel])
            cw = pltpu.async_copy(
                w_hbm.at[idx_slice], wbufs[sel].at[pl.ds(0, ROWS)], sws[sel])
            return cv, cw

        lane = lax.iota(jnp.int32, L)

        def compute_chunk(g):
            sel = g % 2
            gb, wb = gbufs[sel], wbufs[sel]

            @pl.loop(0, NGRP)
            def _(grp):
                gbase = grp * (L * F)
                acc = None
                for j in range(L):
                    r0 = gbase + j * F
                    s = gb[r0]
                    ss = s * s
                    for f in range(1, F):
                        row = gb[r0 + f]
                        s += row
                        ss += row * row
                    o = (j * F) % 8
                    a0 = gbase + (j * F - o)
                    wa = wb[pl.ds(a0, L)]
                    wc = wb[pl.ds(a0 + L, L)]
                    mask_a = (lane >= o).astype(jnp.float32)
                    mask_b = (lane < (o + F - L)).astype(jnp.float32)
                    c = 0.5 * (s * s - ss) + wa * mask_a + wc * mask_b
                    onehot_j = (lane == j).astype(jnp.float32)
                    term = onehot_j * jnp.sum(c)
                    acc = term if acc is None else acc + term
                tbuf[pl.ds(g * E_CH + grp * L, L)] = acc

        pending = start_gather(0)
        for g in range(NCHUNK):
            cv, cw = pending
            cv.wait()
            cw.wait()
            if g + 1 < NCHUNK:
                pending = start_gather(g + 1)
            compute_chunk(g)

        bias = bbuf[pl.ds(0, L)]

        @pl.loop(0, EPW, step=L)
        def _(i):
            logits = tbuf[pl.ds(i, L)] + bias
            obuf[pl.ds(i, L)] = 1.0 / (1.0 + jnp.exp(-logits))

        pltpu.sync_copy(obuf, out_hbm.at[pl.ds(wid * EPW, EPW)])

    return fm_kernel(idx_flat, w1d, vp, b16)


def kernel(inputs, w, v, b):
    B, F = inputs.shape
    N, K = v.shape
    idx_flat = inputs.reshape(B * F)
    w1d = w.reshape(-1)
    b16 = jnp.broadcast_to(b, (L,))
    vp = lax.optimization_barrier(v.reshape(N * K // 128, 128))
    vsc = vp.reshape(N, K)
    return _fm_sc(idx_flat, w1d, vsc, b16, B=B, F=F, N=N, K=K)


# speedup vs baseline: 1.1840x; 1.1840x over previous
import dataclasses
import functools

import jax
import jax.numpy as jnp
import numpy as np
from jax import lax
from jax.experimental import pallas as pl
from jax.experimental.pallas import tpu as pltpu
from jax.experimental.pallas import tpu_sc as plsc

NC, NS, L = 2, 16, 16
NW = NC * NS


def _fm_sc(idx_flat, w1d, vp, b16, *, B, F, N, K):
    EPW = B // NW
    IPW = EPW * F
    E_CH = 64
    ROWS = E_CH * F
    NCHUNK = EPW // E_CH
    NGRP = E_CH // L

    mesh = plsc.VectorSubcoreMesh(
        core_axis_name="c", subcore_axis_name="s", num_cores=NC, num_subcores=NS
    )

    cp = pltpu.CompilerParams(use_tc_tiling_on_sc=False)
    if "needs_layout_passes" in pltpu.CompilerParams.__dataclass_fields__:
        cp = dataclasses.replace(cp, needs_layout_passes=False)

    @functools.partial(
        pl.kernel,
        out_type=jax.ShapeDtypeStruct((B,), jnp.float32),
        mesh=mesh,
        compiler_params=cp,
        scratch_types=[
            pltpu.VMEM((IPW,), jnp.int32),
            pltpu.VMEM((ROWS, K), jnp.float32),
            pltpu.VMEM((ROWS, K), jnp.float32),
            pltpu.VMEM((ROWS + L,), jnp.float32),
            pltpu.VMEM((ROWS + L,), jnp.float32),
            pltpu.VMEM((EPW,), jnp.float32),
            pltpu.VMEM((EPW,), jnp.float32),
            pltpu.VMEM((L,), jnp.float32),
            pltpu.SemaphoreType.DMA,
            pltpu.SemaphoreType.DMA,
            pltpu.SemaphoreType.DMA,
            pltpu.SemaphoreType.DMA,
        ],
    )
    def fm_kernel(idx_hbm, w_hbm, v_hbm, b_hbm, out_hbm,
                  idxv, g0, g1, w0, w1, tbuf, obuf, bbuf,
                  sv0, sv1, sw0, sw1):
        wid = lax.axis_index("s") * NC + lax.axis_index("c")

        pltpu.sync_copy(idx_hbm.at[pl.ds(wid * IPW, IPW)], idxv)
        pltpu.sync_copy(b_hbm, bbuf)

        gbufs, wbufs, svs, sws = (g0, g1), (w0, w1), (sv0, sv1), (sw0, sw1)

        def start_gather(g):
            sel = g % 2
            idx_slice = idxv.at[pl.ds(g * ROWS, ROWS)]
            cv = pltpu.async_copy(v_hbm.at[idx_slice], gbufs[sel], svs[sel])
            cw = pltpu.async_copy(
                w_hbm.at[idx_slice], wbufs[sel].at[pl.ds(0, ROWS)], sws[sel])
            return cv, cw

        lane = lax.iota(jnp.int32, L)

        def compute_chunk(g):
            sel = g % 2
            gb, wb = gbufs[sel], wbufs[sel]

            @pl.loop(0, NGRP)
            def _(grp):
                gbase = grp * (L * F)
                acc = None
                for j in range(L):
                    r0 = gbase + j * F
                    s = gb[r0]
                    ss = s * s
                    for f in range(1, F):
                        row = gb[r0 + f]
                        s += row
                        ss += row * row
                    o = (j * F) % 8
                    a0 = gbase + (j * F - o)
                    wa = wb[pl.ds(a0, L)]
                    wc = wb[pl.ds(a0 + L, L)]
                    mask_a = (lane >= o).astype(jnp.float32)
                    mask_b = (lane < (o + F - L)).astype(jnp.float32)
                    c = 0.5 * (s * s - ss) + wa * mask_a + wc * mask_b
                    onehot_j = (lane == j).astype(jnp.float32)
                    term = onehot_j * jnp.sum(c)
                    acc = term if acc is None else acc + term
                tbuf[pl.ds(g * E_CH + grp * L, L)] = acc

        pending = start_gather(0)
        for g in range(NCHUNK):
            cv, cw = pending
            cv.wait()
            cw.wait()
            if g + 1 < NCHUNK:
                pending = start_gather(g + 1)
            compute_chunk(g)

        bias = bbuf[pl.ds(0, L)]

        @pl.loop(0, EPW, step=L)
        def _(i):
            logits = tbuf[pl.ds(i, L)] + bias
            obuf[pl.ds(i, L)] = 1.0 / (1.0 + jnp.exp(-logits))

        pltpu.sync_copy(obuf, out_hbm.at[pl.ds(wid * EPW, EPW)])

    return fm_kernel(idx_flat, w1d, vp, b16)


def _retile_v(vt, *, N, K, RB=8192):
    G = K // 8

    def body(x_ref, o_ref):
        z3 = x_ref[...].T.reshape(RB // 8, 8, K)
        o_ref[...] = jnp.concatenate(
            [z3[:, m, :] for m in range(8)], axis=1)

    return pl.pallas_call(
        body,
        grid=(pl.cdiv(N, RB),),
        in_specs=[pl.BlockSpec((K, RB), lambda i: (0, i))],
        out_specs=pl.BlockSpec((RB * K // 128, 128), lambda i: (i, 0)),
        out_shape=jax.ShapeDtypeStruct((N * K // 128, 128), jnp.float32),
    )(vt)


def kernel(inputs, w, v, b):
    B, F = inputs.shape
    N, K = v.shape
    idx_flat = inputs.reshape(B * F)
    w1d = w.reshape(-1)
    b16 = jnp.broadcast_to(b, (L,))
    vp = _retile_v(v.T, N=N, K=K)
    vsc = vp.reshape(N, K)
    return _fm_sc(idx_flat, w1d, vsc, b16, B=B, F=F, N=N, K=K)


# speedup vs baseline: 1.4527x; 1.2269x over previous
import dataclasses
import functools

import jax
import jax.numpy as jnp
import numpy as np
from jax import lax
from jax.experimental import pallas as pl
from jax.experimental.pallas import tpu as pltpu
from jax.experimental.pallas import tpu_sc as plsc

NC, NS, L = 2, 16, 16
NW = NC * NS


def _fm_sc(idx_flat, w1d, vp, b16, *, B, F, N, K):
    EPW = B // NW
    IPW = EPW * F
    E_CH = 64
    ROWS = E_CH * F
    NCHUNK = EPW // E_CH
    NGRP = E_CH // L

    mesh = plsc.VectorSubcoreMesh(
        core_axis_name="c", subcore_axis_name="s", num_cores=NC, num_subcores=NS
    )

    cp = pltpu.CompilerParams(use_tc_tiling_on_sc=False)
    if "needs_layout_passes" in pltpu.CompilerParams.__dataclass_fields__:
        cp = dataclasses.replace(cp, needs_layout_passes=False)

    @functools.partial(
        pl.kernel,
        out_type=jax.ShapeDtypeStruct((B,), jnp.float32),
        mesh=mesh,
        compiler_params=cp,
        scratch_types=[
            pltpu.VMEM((IPW,), jnp.int32),
            pltpu.VMEM((IPW,), jnp.int32),
            pltpu.VMEM((ROWS, K), jnp.float32),
            pltpu.VMEM((ROWS, K), jnp.float32),
            pltpu.VMEM((ROWS + L,), jnp.float32),
            pltpu.VMEM((ROWS + L,), jnp.float32),
            pltpu.VMEM((EPW,), jnp.float32),
            pltpu.VMEM((EPW,), jnp.float32),
            pltpu.VMEM((L,), jnp.float32),
            pltpu.SemaphoreType.DMA,
            pltpu.SemaphoreType.DMA,
            pltpu.SemaphoreType.DMA,
            pltpu.SemaphoreType.DMA,
        ],
    )
    def fm_kernel(idx_hbm, w_hbm, v_hbm, b_hbm, out_hbm,
                  idxv, idxp, g0, g1, w0, w1, tbuf, obuf, bbuf,
                  sv0, sv1, sw0, sw1):
        wid = lax.axis_index("s") * NC + lax.axis_index("c")

        pltpu.sync_copy(idx_hbm.at[pl.ds(wid * IPW, IPW)], idxv)
        pltpu.sync_copy(b_hbm, bbuf)

        gbufs, wbufs, svs, sws = (g0, g1), (w0, w1), (sv0, sv1), (sw0, sw1)

        def start_gather(g):
            sel = g % 2
            @pl.loop(g * ROWS, (g + 1) * ROWS, step=L)
            def _(i):
                r = idxv[pl.ds(i, L)]
                idxp[pl.ds(i, L)] = (
                    (r & -64) | ((r & 7) << 3) | ((r >> 3) & 7))

            idx_slice = idxp.at[pl.ds(g * ROWS, ROWS)]
            cv = pltpu.async_copy(v_hbm.at[idx_slice], gbufs[sel], svs[sel])
            cw = pltpu.async_copy(
                w_hbm.at[idxv.at[pl.ds(g * ROWS, ROWS)]],
                wbufs[sel].at[pl.ds(0, ROWS)], sws[sel])
            return cv, cw

        lane = lax.iota(jnp.int32, L)

        def compute_chunk(g):
            sel = g % 2
            gb, wb = gbufs[sel], wbufs[sel]

            @pl.loop(0, NGRP)
            def _(grp):
                gbase = grp * (L * F)
                acc = None
                for j in range(L):
                    r0 = gbase + j * F
                    s = gb[r0]
                    ss = s * s
                    for f in range(1, F):
                        row = gb[r0 + f]
                        s += row
                        ss += row * row
                    o = (j * F) % 8
                    a0 = gbase + (j * F - o)
                    wa = wb[pl.ds(a0, L)]
                    wc = wb[pl.ds(a0 + L, L)]
                    mask_a = (lane >= o).astype(jnp.float32)
                    mask_b = (lane < (o + F - L)).astype(jnp.float32)
                    c = 0.5 * (s * s - ss) + wa * mask_a + wc * mask_b
                    onehot_j = (lane == j).astype(jnp.float32)
                    term = onehot_j * jnp.sum(c)
                    acc = term if acc is None else acc + term
                tbuf[pl.ds(g * E_CH + grp * L, L)] = acc

        pending = start_gather(0)
        for g in range(NCHUNK):
            cv, cw = pending
            cv.wait()
            cw.wait()
            if g + 1 < NCHUNK:
                pending = start_gather(g + 1)
            compute_chunk(g)

        bias = bbuf[pl.ds(0, L)]

        @pl.loop(0, EPW, step=L)
        def _(i):
            logits = tbuf[pl.ds(i, L)] + bias
            obuf[pl.ds(i, L)] = 1.0 / (1.0 + jnp.exp(-logits))

        pltpu.sync_copy(obuf, out_hbm.at[pl.ds(wid * EPW, EPW)])

    return fm_kernel(idx_flat, w1d, vp, b16)


def _retile_v(vt, wt, *, N, K, RB=8192):
    def body(x_ref, w_ref, o_ref, ow_ref):
        z = x_ref[...].T
        zp = jnp.pad(z, ((0, 0), (0, 128 - K)))
        z4 = zp.reshape(RB // 64, 8, 8, 128)
        y = z4[:, 0, :, :]
        for m in range(1, 8):
            y = y + jnp.roll(z4[:, m, :, :], K * m, axis=-1)
        o_ref[...] = y.reshape(RB // 8, 128)
        ow_ref[...] = w_ref[0, :]

    return pl.pallas_call(
        body,
        grid=(pl.cdiv(N, RB),),
        in_specs=[pl.BlockSpec((K, RB), lambda i: (0, i)),
                  pl.BlockSpec((1, RB), lambda i: (0, i))],
        out_specs=[pl.BlockSpec((RB * K // 128, 128), lambda i: (i, 0)),
                   pl.BlockSpec((RB,), lambda i: (i,))],
        out_shape=[jax.ShapeDtypeStruct((N * K // 128, 128), jnp.float32),
                   jax.ShapeDtypeStruct((N,), jnp.float32)],
    )(vt, wt)


def kernel(inputs, w, v, b):
    B, F = inputs.shape
    N, K = v.shape
    idx_flat = inputs.reshape(B * F)
    b16 = jnp.broadcast_to(b, (L,))
    vp, w1d = _retile_v(v.T, w.T, N=N, K=K)
    vsc = vp.reshape(N, K)
    return _fm_sc(idx_flat, w1d, vsc, b16, B=B, F=F, N=N, K=K)


# speedup vs baseline: 2.8734x; 1.9779x over previous
import dataclasses
import functools

import jax
import jax.numpy as jnp
import numpy as np
from jax import lax
from jax.experimental import pallas as pl
from jax.experimental.pallas import tpu as pltpu
from jax.experimental.pallas import tpu_sc as plsc

NC, NS, L = 2, 16, 16
NW = NC * NS


def _fm_sc(idx_flat, w1d, vp, b16, *, B, F, N, K):
    EPW = B // NW
    IPW = EPW * F
    E_CH = 64
    ROWS = E_CH * F
    NCHUNK = EPW // E_CH
    NGRP = E_CH // L

    mesh = plsc.VectorSubcoreMesh(
        core_axis_name="c", subcore_axis_name="s", num_cores=NC, num_subcores=NS
    )

    cp = pltpu.CompilerParams(use_tc_tiling_on_sc=False)
    if "needs_layout_passes" in pltpu.CompilerParams.__dataclass_fields__:
        cp = dataclasses.replace(cp, needs_layout_passes=False)

    @functools.partial(
        pl.kernel,
        out_type=jax.ShapeDtypeStruct((B,), jnp.float32),
        mesh=mesh,
        compiler_params=cp,
        scratch_types=[
            pltpu.VMEM((IPW,), jnp.int32),
            pltpu.VMEM((IPW,), jnp.int32),
            pltpu.VMEM((ROWS, K), jnp.float32),
            pltpu.VMEM((ROWS, K), jnp.float32),
            pltpu.VMEM((ROWS + L,), jnp.float32),
            pltpu.VMEM((ROWS + L,), jnp.float32),
            pltpu.VMEM((EPW,), jnp.float32),
            pltpu.VMEM((EPW,), jnp.float32),
            pltpu.VMEM((L,), jnp.float32),
            pltpu.SemaphoreType.DMA,
            pltpu.SemaphoreType.DMA,
            pltpu.SemaphoreType.DMA,
            pltpu.SemaphoreType.DMA,
        ],
    )
    def fm_kernel(idx_hbm, w_hbm, v_hbm, b_hbm, out_hbm,
                  idxv, idxp, g0, g1, w0, w1, tbuf, obuf, bbuf,
                  sv0, sv1, sw0, sw1):
        wid = lax.axis_index("s") * NC + lax.axis_index("c")

        pltpu.sync_copy(idx_hbm.at[pl.ds(wid * IPW, IPW)], idxv)
        pltpu.sync_copy(b_hbm, bbuf)

        gbufs, wbufs, svs, sws = (g0, g1), (w0, w1), (sv0, sv1), (sw0, sw1)

        def start_gather(g):
            sel = g % 2
            @pl.loop(g * ROWS, (g + 1) * ROWS, step=L)
            def _(i):
                r = idxv[pl.ds(i, L)]
                idxp[pl.ds(i, L)] = (
                    (r & -64) | ((r & 7) << 3) | ((r >> 3) & 7))

            idx_slice = idxp.at[pl.ds(g * ROWS, ROWS)]
            cv = pltpu.async_copy(v_hbm.at[idx_slice], gbufs[sel], svs[sel])
            cw = pltpu.async_copy(
                w_hbm.at[idxv.at[pl.ds(g * ROWS, ROWS)]],
                wbufs[sel].at[pl.ds(0, ROWS)], sws[sel])
            return cv, cw

        lane = lax.iota(jnp.int32, L)

        def compute_chunk(g):
            sel = g % 2
            gb, wb = gbufs[sel], wbufs[sel]

            @pl.loop(0, NGRP)
            def _(grp):
                gbase = grp * (L * F)
                acc = None
                for j in range(L):
                    r0 = gbase + j * F
                    s = gb[r0]
                    ss = s * s
                    for f in range(1, F):
                        row = gb[r0 + f]
                        s += row
                        ss += row * row
                    o = (j * F) % 8
                    a0 = gbase + (j * F - o)
                    wa = wb[pl.ds(a0, L)]
                    wc = wb[pl.ds(a0 + L, L)]
                    mask_a = (lane >= o).astype(jnp.float32)
                    mask_b = (lane < (o + F - L)).astype(jnp.float32)
                    c = 0.5 * (s * s - ss) + wa * mask_a + wc * mask_b
                    onehot_j = (lane == j).astype(jnp.float32)
                    term = onehot_j * jnp.sum(c)
                    acc = term if acc is None else acc + term
                tbuf[pl.ds(g * E_CH + grp * L, L)] = acc

        pending = start_gather(0)
        for g in range(NCHUNK):
            cv, cw = pending
            cv.wait()
            cw.wait()
            if g + 1 < NCHUNK:
                pending = start_gather(g + 1)
            compute_chunk(g)

        bias = bbuf[pl.ds(0, L)]

        @pl.loop(0, EPW, step=L)
        def _(i):
            logits = tbuf[pl.ds(i, L)] + bias
            obuf[pl.ds(i, L)] = 1.0 / (1.0 + jnp.exp(-logits))

        pltpu.sync_copy(obuf, out_hbm.at[pl.ds(wid * EPW, EPW)])

    return fm_kernel(idx_flat, w1d, vp, b16)


def _retile_v(vt, wt, *, N, K, RB=16384):
    def body(x_ref, w_ref, o_ref, ow_ref):
        x = x_ref[...]
        kk = lax.broadcasted_iota(jnp.int32, (K, 128), 0)
        cc = lax.broadcasted_iota(jnp.int32, (K, 128), 1)
        sel = (kk == (cc & (K - 1))).astype(jnp.bfloat16)
        y = lax.dot_general(x.astype(jnp.bfloat16), sel,
                            (((0,), (0,)), ((), ())),
                            preferred_element_type=jnp.float32)
        y4 = y.reshape(RB // 64, 8, 8, 128)
        grp = lax.broadcasted_iota(jnp.int32, (RB // 64, 8, 128), 2) // K
        acc = jnp.where(grp == 0, y4[:, 0, :, :], 0.0)
        for m in range(1, 8):
            acc = acc + jnp.where(grp == m, y4[:, m, :, :], 0.0)
        o_ref[...] = acc.reshape(RB // 8, 128)
        ow_ref[...] = w_ref[0, :]

    return pl.pallas_call(
        body,
        grid=(pl.cdiv(N, RB),),
        in_specs=[pl.BlockSpec((K, RB), lambda i: (0, i)),
                  pl.BlockSpec((1, RB), lambda i: (0, i))],
        out_specs=[pl.BlockSpec((RB * K // 128, 128), lambda i: (i, 0)),
                   pl.BlockSpec((RB,), lambda i: (i,))],
        out_shape=[jax.ShapeDtypeStruct((N * K // 128, 128), jnp.float32),
                   jax.ShapeDtypeStruct((N,), jnp.float32)],
    )(vt, wt)


def kernel(inputs, w, v, b):
    B, F = inputs.shape
    N, K = v.shape
    idx_flat = inputs.reshape(B * F)
    b16 = jnp.broadcast_to(b, (L,))
    vp, w1d = _retile_v(v.T, w.T, N=N, K=K)
    vsc = vp.reshape(N, K)
    return _fm_sc(idx_flat, w1d, vsc, b16, B=B, F=F, N=N, K=K)


# speedup vs baseline: 3.2098x; 1.1171x over previous
import dataclasses
import functools

import jax
import jax.numpy as jnp
import numpy as np
from jax import lax
from jax.experimental import pallas as pl
from jax.experimental.pallas import tpu as pltpu
from jax.experimental.pallas import tpu_sc as plsc

NC, NS, L = 2, 16, 16
NW = NC * NS


def _fm_sc(idx_flat, w1d, vp, b16, *, B, F, N, K):
    EPW = B // NW
    IPW = EPW * F
    E_CH = 64
    ROWS = E_CH * F
    NCHUNK = EPW // E_CH
    NGRP = E_CH // L

    mesh = plsc.VectorSubcoreMesh(
        core_axis_name="c", subcore_axis_name="s", num_cores=NC, num_subcores=NS
    )

    cp = pltpu.CompilerParams(use_tc_tiling_on_sc=False)
    if "needs_layout_passes" in pltpu.CompilerParams.__dataclass_fields__:
        cp = dataclasses.replace(cp, needs_layout_passes=False)

    @functools.partial(
        pl.kernel,
        out_type=jax.ShapeDtypeStruct((B,), jnp.float32),
        mesh=mesh,
        compiler_params=cp,
        scratch_types=[
            pltpu.VMEM((IPW,), jnp.int32),
            pltpu.VMEM((IPW,), jnp.int32),
            pltpu.VMEM((ROWS, K), jnp.float32),
            pltpu.VMEM((ROWS, K), jnp.float32),
            pltpu.VMEM((ROWS + L,), jnp.float32),
            pltpu.VMEM((ROWS + L,), jnp.float32),
            pltpu.VMEM((EPW,), jnp.float32),
            pltpu.VMEM((EPW,), jnp.float32),
            pltpu.VMEM((L,), jnp.float32),
            pltpu.SemaphoreType.DMA,
            pltpu.SemaphoreType.DMA,
            pltpu.SemaphoreType.DMA,
            pltpu.SemaphoreType.DMA,
        ],
    )
    def fm_kernel(idx_hbm, w_hbm, v_hbm, b_hbm, out_hbm,
                  idxv, idxp, g0, g1, w0, w1, tbuf, obuf, bbuf,
                  sv0, sv1, sw0, sw1):
        wid = lax.axis_index("s") * NC + lax.axis_index("c")

        pltpu.sync_copy(idx_hbm.at[pl.ds(wid * IPW, IPW)], idxv)
        pltpu.sync_copy(b_hbm, bbuf)

        gbufs, wbufs, svs, sws = (g0, g1), (w0, w1), (sv0, sv1), (sw0, sw1)

        def start_gather(g):
            sel = g % 2
            @pl.loop(g * ROWS, (g + 1) * ROWS, step=L)
            def _(i):
                r = idxv[pl.ds(i, L)]
                idxp[pl.ds(i, L)] = (
                    (r & -64) | ((r & 7) << 3) | ((r >> 3) & 7))

            idx_slice = idxp.at[pl.ds(g * ROWS, ROWS)]
            cv = pltpu.async_copy(v_hbm.at[idx_slice], gbufs[sel], svs[sel])
            cw = pltpu.async_copy(
                w_hbm.at[idxv.at[pl.ds(g * ROWS, ROWS)]],
                wbufs[sel].at[pl.ds(0, ROWS)], sws[sel])
            return cv, cw

        lane = lax.iota(jnp.int32, L)
        masks_a = [(lane >= o).astype(jnp.float32) for o in range(8)]
        masks_b = [(lane < (o + F - L)).astype(jnp.float32) for o in range(8)]
        onehots = [(lane == j).astype(jnp.float32) for j in range(L)]

        def compute_chunk(g):
            sel = g % 2
            gb, wb = gbufs[sel], wbufs[sel]

            @pl.loop(0, NGRP)
            def _(grp):
                gbase = grp * (L * F)
                acc = None
                for j in range(L):
                    r0 = gbase + j * F
                    s = gb[r0]
                    ss = s * s
                    for f in range(1, F):
                        row = gb[r0 + f]
                        s += row
                        ss += row * row
                    o = (j * F) % 8
                    a0 = gbase + (j * F - o)
                    wa = wb[pl.ds(a0, L)]
                    wc = wb[pl.ds(a0 + L, L)]
                    c = 0.5 * (s * s - ss) + wa * masks_a[o] + wc * masks_b[o]
                    term = onehots[j] * jnp.sum(c)
                    acc = term if acc is None else acc + term
                tbuf[pl.ds(g * E_CH + grp * L, L)] = acc

        pending = start_gather(0)
        for g in range(NCHUNK):
            cv, cw = pending
            cv.wait()
            cw.wait()
            if g + 1 < NCHUNK:
                pending = start_gather(g + 1)
            compute_chunk(g)

        bias = bbuf[pl.ds(0, L)]

        @pl.loop(0, EPW, step=L)
        def _(i):
            logits = tbuf[pl.ds(i, L)] + bias
            obuf[pl.ds(i, L)] = 1.0 / (1.0 + jnp.exp(-logits))

        pltpu.sync_copy(obuf, out_hbm.at[pl.ds(wid * EPW, EPW)])

    return fm_kernel(idx_flat, w1d, vp, b16)


def _retile_v(vt, wt, *, N, K, RB=32768):
    def body(x_ref, w_ref, o_ref, ow_ref):
        x = x_ref[...]
        kk = lax.broadcasted_iota(jnp.int32, (K, 128), 0)
        cc = lax.broadcasted_iota(jnp.int32, (K, 128), 1)
        sel = (kk == (cc & (K - 1))).astype(jnp.bfloat16)
        y = lax.dot_general(x.astype(jnp.bfloat16), sel,
                            (((0,), (0,)), ((), ())),
                            preferred_element_type=jnp.float32)
        y4 = y.reshape(RB // 64, 8, 8, 128)
        grp = lax.broadcasted_iota(jnp.int32, (RB // 64, 8, 128), 2) // K
        acc = jnp.where(grp == 0, y4[:, 0, :, :], 0.0)
        for m in range(1, 8):
            acc = acc + jnp.where(grp == m, y4[:, m, :, :], 0.0)
        o_ref[...] = acc.reshape(RB // 8, 128)
        ow_ref[...] = w_ref[0, :]

    return pl.pallas_call(
        body,
        grid=(pl.cdiv(N, RB),),
        in_specs=[pl.BlockSpec((K, RB), lambda i: (0, i)),
                  pl.BlockSpec((1, RB), lambda i: (0, i))],
        out_specs=[pl.BlockSpec((RB * K // 128, 128), lambda i: (i, 0)),
                   pl.BlockSpec((RB,), lambda i: (i,))],
        out_shape=[jax.ShapeDtypeStruct((N * K // 128, 128), jnp.float32),
                   jax.ShapeDtypeStruct((N,), jnp.float32)],
    )(vt, wt)


def kernel(inputs, w, v, b):
    B, F = inputs.shape
    N, K = v.shape
    idx_flat = inputs.reshape(B * F)
    b16 = jnp.broadcast_to(b, (L,))
    vp, w1d = _retile_v(v.T, w.T, N=N, K=K)
    vsc = vp.reshape(N, K)
    return _fm_sc(idx_flat, w1d, vsc, b16, B=B, F=F, N=N, K=K)
